# Initial kernel scaffold; baseline (speedup 1.0000x reference)
import functools

import numpy as np
import jax
import jax.numpy as jnp
from jax import lax
from jax.experimental import pallas as pl
from jax.experimental.pallas import tpu as pltpu
from jax.experimental.pallas import tpu_sc as plsc

_NUM_USERS = 50000
_NUM_ITEMS = 50000
_N_NODES = _NUM_USERS + _NUM_ITEMS
_DROP = 0.1
_E = 6400000

_NC = 2
_NS = 16
_NW = _NC * _NS
_EPW = _E // _NW
_C = 2000
_NCHUNK = _EPW // _C
_VPC = _C // 16


@functools.lru_cache(maxsize=None)
def _keep_f32() -> np.ndarray:
    ku, ki = jax.random.split(jax.random.key(42))
    user_perm = np.asarray(jax.random.permutation(ku, _NUM_USERS))
    item_perm = np.asarray(jax.random.permutation(ki, _NUM_ITEMS))
    flag = np.zeros((_N_NODES,), np.bool_)
    flag[user_perm[: int(_NUM_USERS * _DROP)]] = True
    flag[item_perm[: int(_NUM_ITEMS * _DROP)] + _NUM_USERS] = True
    return (~flag).astype(np.float32)


_MESH = plsc.VectorSubcoreMesh(core_axis_name="c", subcore_axis_name="s")


@functools.partial(
    pl.kernel,
    out_type=jax.ShapeDtypeStruct((_E,), jnp.float32),
    mesh=_MESH,
    scratch_types=[
        pltpu.VMEM((_N_NODES,), jnp.float32),
        pltpu.VMEM((_C,), jnp.int32),
        pltpu.VMEM((_C,), jnp.int32),
        pltpu.VMEM((_C,), jnp.float32),
        pltpu.VMEM((_C,), jnp.float32),
    ],
)
def _node_dropout_sc(keep_hbm, edge_hbm, vals_hbm, out_hbm,
                     keep_v, src_v, dst_v, vals_v, out_v):
    wid = lax.axis_index("s") * _NC + lax.axis_index("c")
    base_w = wid * _EPW
    pltpu.sync_copy(keep_hbm, keep_v)

    def chunk_body(c, carry):
        base = base_w + c * _C
        pltpu.sync_copy(edge_hbm.at[0, pl.ds(base, _C)], src_v)
        pltpu.sync_copy(edge_hbm.at[1, pl.ds(base, _C)], dst_v)
        pltpu.sync_copy(vals_hbm.at[pl.ds(base, _C)], vals_v)

        def vec_body(i, carry2):
            sl = pl.ds(i * 16, 16)
            ks = plsc.load_gather(keep_v, [src_v[sl]])
            kd = plsc.load_gather(keep_v, [dst_v[sl]])
            out_v[sl] = vals_v[sl] * ks * kd
            return carry2

        lax.fori_loop(0, _VPC, vec_body, 0)
        pltpu.sync_copy(out_v, out_hbm.at[pl.ds(base, _C)])
        return carry

    lax.fori_loop(0, _NCHUNK, chunk_body, 0)


def kernel(edge_index, values):
    keep = jnp.asarray(_keep_f32())
    return _node_dropout_sc(keep, edge_index, values)


# speedup vs baseline: 362.5550x; 362.5550x over previous
import functools

import numpy as np
import jax
import jax.numpy as jnp
from jax import lax
from jax.experimental import pallas as pl
from jax.experimental.pallas import tpu as pltpu
from jax.experimental.pallas import tpu_sc as plsc

_NUM_USERS = 50000
_NUM_ITEMS = 50000
_N_NODES = _NUM_USERS + _NUM_ITEMS
_DROP = 0.1
_E = 6400000

_NC = 2
_NS = 16
_NW = _NC * _NS
_C = 2048
_NCHUNK = _E // _C
_VPC = _C // 16


_N_DROP_U = int(_NUM_USERS * _DROP)
_N_DROP_I = int(_NUM_ITEMS * _DROP)
_KEEP_CACHE: list = []


def _keep_f32() -> jax.Array:
    if _KEEP_CACHE:
        return jnp.asarray(_KEEP_CACHE[0])
    try:
        with jax.ensure_compile_time_eval():
            ku, ki = jax.random.split(jax.random.key(42))
            user_perm = np.asarray(jax.random.permutation(ku, _NUM_USERS))
            item_perm = np.asarray(jax.random.permutation(ki, _NUM_ITEMS))
        flag = np.zeros((_N_NODES,), np.bool_)
        flag[user_perm[:_N_DROP_U]] = True
        flag[item_perm[:_N_DROP_I] + _NUM_USERS] = True
        keep = (~flag).astype(np.float32)
        _KEEP_CACHE.append(keep)
        return jnp.asarray(keep)
    except Exception:
        ku, ki = jax.random.split(jax.random.key(42))
        user_perm = jax.random.permutation(ku, _NUM_USERS)
        item_perm = jax.random.permutation(ki, _NUM_ITEMS)
        flag = jnp.zeros((_N_NODES,), bool)
        flag = flag.at[user_perm[:_N_DROP_U]].set(True)
        flag = flag.at[item_perm[:_N_DROP_I] + _NUM_USERS].set(True)
        return (~flag).astype(jnp.float32)


_MESH = plsc.VectorSubcoreMesh(core_axis_name="c", subcore_axis_name="s")


@functools.partial(
    pl.kernel,
    out_type=jax.ShapeDtypeStruct((_E,), jnp.float32),
    mesh=_MESH,
    compiler_params=pltpu.CompilerParams(needs_layout_passes=False),
    scratch_types=[
        pltpu.VMEM((_N_NODES,), jnp.float32),
        pltpu.VMEM((2, _C), jnp.int32),
        pltpu.VMEM((_C,), jnp.float32),
        pltpu.VMEM((_C,), jnp.float32),
    ],
)
def _node_dropout_sc(keep_hbm, edge_hbm, vals_hbm, out_hbm,
                     keep_v, edge_v, vals_v, out_v):
    wid = lax.axis_index("s") * _NC + lax.axis_index("c")
    pltpu.sync_copy(keep_hbm, keep_v)
    my_chunks = (_NCHUNK - wid + _NW - 1) // _NW

    def chunk_body(j, carry):
        base = (wid + j * _NW) * _C
        pltpu.sync_copy(edge_hbm.at[:, pl.ds(base, _C)], edge_v)
        pltpu.sync_copy(vals_hbm.at[pl.ds(base, _C)], vals_v)

        def vec_body(i, carry2):
            sl = pl.ds(i * 16, 16)
            ks = plsc.load_gather(keep_v, [edge_v[0, sl]])
            kd = plsc.load_gather(keep_v, [edge_v[1, sl]])
            out_v[sl] = vals_v[sl] * ks * kd
            return carry2

        lax.fori_loop(0, _VPC, vec_body, 0)
        pltpu.sync_copy(out_v, out_hbm.at[pl.ds(base, _C)])
        return carry

    lax.fori_loop(0, my_chunks, chunk_body, 0)


def kernel(edge_index, values):
    keep = jnp.asarray(_keep_f32())
    return _node_dropout_sc(keep, edge_index, values)


# speedup vs baseline: 1338.2606x; 3.6912x over previous
import functools

import numpy as np
import jax
import jax.numpy as jnp
from jax import lax
from jax.experimental import pallas as pl
from jax.experimental.pallas import tpu as pltpu
from jax.experimental.pallas import tpu_sc as plsc

_NUM_USERS = 50000
_NUM_ITEMS = 50000
_N_NODES = _NUM_USERS + _NUM_ITEMS
_DROP = 0.1
_E = 6400000

_NC = 2
_NS = 16
_NW = _NC * _NS
_C = 10240
_NCHUNK = _E // _C
_MAXJ = (_NCHUNK + _NW - 1) // _NW
_VPC = _C // 16
_NWORDS = _N_NODES // 32
_NWORDS_PAD = 3200

_N_DROP_U = int(_NUM_USERS * _DROP)
_N_DROP_I = int(_NUM_ITEMS * _DROP)
_KEEP_CACHE: list = []


def _keep_words() -> jax.Array:
    if _KEEP_CACHE:
        return jnp.asarray(_KEEP_CACHE[0])
    try:
        with jax.ensure_compile_time_eval():
            ku, ki = jax.random.split(jax.random.key(42))
            user_perm = np.asarray(jax.random.permutation(ku, _NUM_USERS))
            item_perm = np.asarray(jax.random.permutation(ki, _NUM_ITEMS))
        flag = np.zeros((_N_NODES,), np.bool_)
        flag[user_perm[:_N_DROP_U]] = True
        flag[item_perm[:_N_DROP_I] + _NUM_USERS] = True
        idx = np.arange(_N_NODES)
        words = np.zeros((_NWORDS_PAD,), np.uint32)
        np.bitwise_or.at(
            words, idx >> 5,
            (~flag).astype(np.uint32) << (idx & 31).astype(np.uint32))
        packed = words.view(np.int32)
        _KEEP_CACHE.append(packed)
        return jnp.asarray(packed)
    except Exception:
        ku, ki = jax.random.split(jax.random.key(42))
        user_perm = jax.random.permutation(ku, _NUM_USERS)
        item_perm = jax.random.permutation(ki, _NUM_ITEMS)
        flag = jnp.zeros((_N_NODES,), bool)
        flag = flag.at[user_perm[:_N_DROP_U]].set(True)
        flag = flag.at[item_perm[:_N_DROP_I] + _NUM_USERS].set(True)
        lanes = (~flag).reshape(_NWORDS, 32).astype(jnp.uint32)
        words = jnp.sum(lanes << jnp.arange(32, dtype=jnp.uint32)[None, :],
                        axis=1, dtype=jnp.uint32)
        words = jnp.concatenate(
            [words, jnp.zeros((_NWORDS_PAD - _NWORDS,), jnp.uint32)])
        return lax.bitcast_convert_type(words, jnp.int32)


_MESH = plsc.VectorSubcoreMesh(core_axis_name="c", subcore_axis_name="s")


@functools.partial(
    pl.kernel,
    out_type=jax.ShapeDtypeStruct((_E,), jnp.float32),
    mesh=_MESH,
    compiler_params=pltpu.CompilerParams(needs_layout_passes=False),
    scratch_types=[
        pltpu.VMEM((_NWORDS_PAD,), jnp.int32),
        pltpu.VMEM((2, 2, _C), jnp.int32),
        pltpu.VMEM((2, _C), jnp.float32),
        pltpu.VMEM((2, _C), jnp.float32),
        pltpu.SemaphoreType.DMA,
        pltpu.SemaphoreType.DMA,
        pltpu.SemaphoreType.DMA,
        pltpu.SemaphoreType.DMA,
    ],
)
def _node_dropout_sc(kw_hbm, edge_hbm, vals_hbm, out_hbm,
                     kw_v, e_v, v_v, o_v, isem0, isem1, osem0, osem1):
    wid = lax.axis_index("s") * _NC + lax.axis_index("c")
    pltpu.sync_copy(kw_hbm, kw_v)
    isems = (isem0, isem1)
    osems = (osem0, osem1)

    def issue_in(j, b):
        @pl.when(wid + j * _NW < _NCHUNK)
        def _():
            base = (wid + j * _NW) * _C
            pltpu.async_copy(edge_hbm.at[:, pl.ds(base, _C)], e_v.at[b],
                             isems[b])
            pltpu.async_copy(vals_hbm.at[pl.ds(base, _C)], v_v.at[b],
                             isems[b])

    def wait_in(j, b):
        base = (wid + j * _NW) * _C
        pltpu.make_async_copy(edge_hbm.at[:, pl.ds(base, _C)], e_v.at[b],
                              isems[b]).wait()
        pltpu.make_async_copy(vals_hbm.at[pl.ds(base, _C)], v_v.at[b],
                              isems[b]).wait()

    def issue_out(j, b):
        base = (wid + j * _NW) * _C
        pltpu.async_copy(o_v.at[b], out_hbm.at[pl.ds(base, _C)], osems[b])

    def wait_out(j, b):
        base = (wid + j * _NW) * _C
        pltpu.make_async_copy(o_v.at[b], out_hbm.at[pl.ds(base, _C)],
                              osems[b]).wait()

    def compute(b):
        @plsc.parallel_loop(0, _VPC, unroll=8)
        def _(i):
            sl = pl.ds(i * 16, 16)
            s = e_v[b, 0, sl]
            d = e_v[b, 1, sl]
            ws = plsc.load_gather(kw_v, [lax.shift_right_logical(s, 5)])
            wd = plsc.load_gather(kw_v, [lax.shift_right_logical(d, 5)])
            m = (lax.shift_right_logical(ws, s & 31)
                 & lax.shift_right_logical(wd, d & 31) & 1)
            o_v[b, sl] = jnp.where(m != 0, v_v[b, sl], 0.0)

    issue_in(0, 0)

    def dstep(jj, carry):
        for b in (0, 1):
            j = jj * 2 + b

            @pl.when(wid + j * _NW < _NCHUNK)
            def _(j=j, b=b):
                wait_in(j, b)
                issue_in(j + 1, 1 - b)

                @pl.when(j >= 2)
                def _():
                    wait_out(j - 2, b)

                compute(b)
                issue_out(j, b)

        return carry

    lax.fori_loop(0, _MAXJ // 2, dstep, 0)

    wait_out(0, 0)
    wait_out(1, 1)


def kernel(edge_index, values):
    kw = _keep_words()
    return _node_dropout_sc(kw, edge_index, values)


# speedup vs baseline: 1838.7019x; 1.3739x over previous
import functools

import numpy as np
import jax
import jax.numpy as jnp
from jax import lax
from jax.experimental import pallas as pl
from jax.experimental.pallas import tpu as pltpu
from jax.experimental.pallas import tpu_sc as plsc

_NUM_USERS = 50000
_NUM_ITEMS = 50000
_N_NODES = _NUM_USERS + _NUM_ITEMS
_DROP = 0.1
_E = 6400000

_NC = 2
_NS = 16
_NW = _NC * _NS
_C = 5120
_NCHUNK = _E // _C
_NBUF = 3
_MAXJ = (_NCHUNK + _NW - 1) // _NW
_MAXJ_PAD = -(-_MAXJ // _NBUF) * _NBUF
_VPC = _C // 16
_NWORDS = _N_NODES // 32
_NWORDS_PAD = 3200

_N_DROP_U = int(_NUM_USERS * _DROP)
_N_DROP_I = int(_NUM_ITEMS * _DROP)
_KEEP_CACHE: list = []


def _keep_words() -> jax.Array:
    if _KEEP_CACHE:
        return jnp.asarray(_KEEP_CACHE[0])
    try:
        with jax.ensure_compile_time_eval():
            ku, ki = jax.random.split(jax.random.key(42))
            user_perm = np.asarray(jax.random.permutation(ku, _NUM_USERS))
            item_perm = np.asarray(jax.random.permutation(ki, _NUM_ITEMS))
        flag = np.zeros((_N_NODES,), np.bool_)
        flag[user_perm[:_N_DROP_U]] = True
        flag[item_perm[:_N_DROP_I] + _NUM_USERS] = True
        idx = np.arange(_N_NODES)
        words = np.zeros((_NWORDS_PAD,), np.uint32)
        np.bitwise_or.at(
            words, idx >> 5,
            (~flag).astype(np.uint32) << (idx & 31).astype(np.uint32))
        packed = words.view(np.int32)
        _KEEP_CACHE.append(packed)
        return jnp.asarray(packed)
    except Exception:
        ku, ki = jax.random.split(jax.random.key(42))
        user_perm = jax.random.permutation(ku, _NUM_USERS)
        item_perm = jax.random.permutation(ki, _NUM_ITEMS)
        flag = jnp.zeros((_N_NODES,), bool)
        flag = flag.at[user_perm[:_N_DROP_U]].set(True)
        flag = flag.at[item_perm[:_N_DROP_I] + _NUM_USERS].set(True)
        lanes = (~flag).reshape(_NWORDS, 32).astype(jnp.uint32)
        words = jnp.sum(lanes << jnp.arange(32, dtype=jnp.uint32)[None, :],
                        axis=1, dtype=jnp.uint32)
        words = jnp.concatenate(
            [words, jnp.zeros((_NWORDS_PAD - _NWORDS,), jnp.uint32)])
        return lax.bitcast_convert_type(words, jnp.int32)


_MESH = plsc.VectorSubcoreMesh(core_axis_name="c", subcore_axis_name="s")


@functools.partial(
    pl.kernel,
    out_type=jax.ShapeDtypeStruct((_E,), jnp.float32),
    mesh=_MESH,
    compiler_params=pltpu.CompilerParams(needs_layout_passes=False),
    scratch_types=[
        pltpu.VMEM((_NWORDS_PAD,), jnp.int32),
        pltpu.VMEM((2, _C), jnp.int32),
        pltpu.VMEM((2, _C), jnp.int32),
        pltpu.VMEM((2, _C), jnp.int32),
        pltpu.VMEM((_C,), jnp.float32),
        pltpu.VMEM((_C,), jnp.float32),
        pltpu.VMEM((_C,), jnp.float32),
        pltpu.VMEM((_C,), jnp.float32),
        pltpu.VMEM((_C,), jnp.float32),
        pltpu.VMEM((_C,), jnp.float32),
        pltpu.SemaphoreType.DMA,
        pltpu.SemaphoreType.DMA,
        pltpu.SemaphoreType.DMA,
        pltpu.SemaphoreType.DMA,
        pltpu.SemaphoreType.DMA,
        pltpu.SemaphoreType.DMA,
    ],
)
def _node_dropout_sc(kw_hbm, edge_hbm, vals_hbm, out_hbm,
                     kw_v, e_v0, e_v1, e_v2, v_v0, v_v1, v_v2,
                     o_v0, o_v1, o_v2,
                     isem0, isem1, isem2, osem0, osem1, osem2):
    wid = lax.axis_index("s") * _NC + lax.axis_index("c")
    pltpu.sync_copy(kw_hbm, kw_v)
    e_bufs = (e_v0, e_v1, e_v2)
    v_bufs = (v_v0, v_v1, v_v2)
    o_bufs = (o_v0, o_v1, o_v2)
    isems = (isem0, isem1, isem2)
    osems = (osem0, osem1, osem2)

    def issue_in(j, b):
        @pl.when(wid + j * _NW < _NCHUNK)
        def _():
            base = (wid + j * _NW) * _C
            pltpu.async_copy(edge_hbm.at[:, pl.ds(base, _C)], e_bufs[b],
                             isems[b])
            pltpu.async_copy(vals_hbm.at[pl.ds(base, _C)], v_bufs[b],
                             isems[b])

    def wait_in(j, b):
        base = (wid + j * _NW) * _C
        pltpu.make_async_copy(edge_hbm.at[:, pl.ds(base, _C)], e_bufs[b],
                              isems[b]).wait()
        pltpu.make_async_copy(vals_hbm.at[pl.ds(base, _C)], v_bufs[b],
                              isems[b]).wait()

    def issue_out(j, b):
        base = (wid + j * _NW) * _C
        pltpu.async_copy(o_bufs[b], out_hbm.at[pl.ds(base, _C)], osems[b])

    def wait_out(j, b):
        base = (wid + j * _NW) * _C
        pltpu.make_async_copy(o_bufs[b], out_hbm.at[pl.ds(base, _C)],
                              osems[b]).wait()

    def compute(b):
        e_v, v_v, o_v = e_bufs[b], v_bufs[b], o_bufs[b]

        @plsc.parallel_loop(0, _VPC, unroll=16)
        def _(i):
            sl = pl.ds(i * 16, 16)
            s = e_v[0, sl]
            d = e_v[1, sl]
            ws = plsc.load_gather(kw_v, [lax.shift_right_logical(s, 5)])
            wd = plsc.load_gather(kw_v, [lax.shift_right_logical(d, 5)])
            m = (lax.shift_right_logical(ws, s & 31)
                 & lax.shift_right_logical(wd, d & 31) & 1)
            o_v[sl] = jnp.where(m != 0, v_v[sl], 0.0)

    issue_in(0, 0)
    issue_in(1, 1)

    def dstep(jj, carry):
        for b in range(_NBUF):
            j = jj * _NBUF + b

            @pl.when(wid + j * _NW < _NCHUNK)
            def _(j=j, b=b):
                wait_in(j, b)
                issue_in(j + 2, (b + 2) % _NBUF)

                @pl.when(j >= _NBUF)
                def _():
                    wait_out(j - _NBUF, b)

                compute(b)
                issue_out(j, b)

        return carry

    lax.fori_loop(0, _MAXJ_PAD // _NBUF, dstep, 0)

    for b in range(_NBUF):
        wait_out(b, b)


def kernel(edge_index, values):
    kw = _keep_words()
    return _node_dropout_sc(kw, edge_index, values)
